# baseline (speedup 1.0000x reference)
import jax
import jax.numpy as jnp
from jax import lax
from jax.experimental import pallas as pl
from jax.experimental.pallas import tpu as pltpu
from jax.experimental.pallas import tpu_sc as plsc

VOCAB = 1000
BATCH = 4096
HIST = 20

_info = plsc.get_sparse_core_info()
NC = _info.num_cores
NS = _info.num_subcores
L = _info.num_lanes
NW = NC * NS
RW = BATCH // NW
C = 32
NCHUNK = RW // C
GPC = C // L
ZUNROLL = 16


def _mh_body(x_hbm, out_hbm, xv, acc0, acc1, sem0, sem1):
    c = lax.axis_index("c")
    s = lax.axis_index("s")
    wid = s * NC + c
    base = wid * RW

    pltpu.sync_copy(x_hbm.at[pl.ds(base * HIST, RW * HIST)], xv)

    zeros = jnp.zeros((L,), jnp.float32)
    ones = jnp.ones((L,), jnp.float32)
    lane_iota = lax.iota(jnp.int32, L)
    lane_row = lane_iota * VOCAB
    lane_pos = lane_iota * HIST

    def _z(i, carry):
        for u in range(ZUNROLL):
            acc0[pl.ds(i * (L * ZUNROLL) + u * L, L)] = zeros
            acc1[pl.ds(i * (L * ZUNROLL) + u * L, L)] = zeros
        return carry

    lax.fori_loop(0, C * VOCAB // (L * ZUNROLL), _z, None)

    accs = (acc0, acc1)
    sems = (sem0, sem1)

    def scatter_chunk(acc, k, val):
        for g in range(GPC):
            rowoff = lane_row + g * (L * VOCAB)
            pos0 = lane_pos + (k * C + g * L) * HIST
            tgts = [
                plsc.load_gather(xv, [pos0 + j]) + rowoff for j in range(HIST)
            ]
            for tgt in tgts:
                if val is None:
                    plsc.addupdate_scatter(acc, [tgt], ones)
                else:
                    plsc.store_scatter(acc, [tgt], val)

    copies = [None] * NCHUNK
    for k in range(NCHUNK):
        acc = accs[k % 2]
        if k >= 2:
            copies[k - 2].wait()
            scatter_chunk(acc, k - 2, zeros)
        scatter_chunk(acc, k, None)
        copies[k] = pltpu.async_copy(
            acc, out_hbm.at[pl.ds((base + k * C) * VOCAB, C * VOCAB)], sems[k % 2]
        )
    copies[NCHUNK - 2].wait()
    copies[NCHUNK - 1].wait()


def kernel(x):
    x_flat = x.reshape(-1)
    out = pl.kernel(
        _mh_body,
        out_type=jax.ShapeDtypeStruct((BATCH * VOCAB,), jnp.float32),
        mesh=plsc.VectorSubcoreMesh(core_axis_name="c", subcore_axis_name="s"),
        scratch_types=[
            pltpu.VMEM((RW * HIST,), jnp.int32),
            pltpu.VMEM((C * VOCAB,), jnp.float32),
            pltpu.VMEM((C * VOCAB,), jnp.float32),
            pltpu.SemaphoreType.DMA,
            pltpu.SemaphoreType.DMA,
        ],
        compiler_params=pltpu.CompilerParams(needs_layout_passes=False),
    )(x_flat)
    return out.reshape(BATCH, VOCAB)


# speedup vs baseline: 1.4123x; 1.4123x over previous
import jax
import jax.numpy as jnp
from jax import lax
from jax.experimental import pallas as pl
from jax.experimental.pallas import tpu as pltpu
from jax.experimental.pallas import tpu_sc as plsc

VOCAB = 1000
BATCH = 4096
HIST = 20

_info = plsc.get_sparse_core_info()
NC = _info.num_cores
NS = _info.num_subcores
L = _info.num_lanes
NW = NC * NS
RW = BATCH // NW
C = 32
NCHUNK = RW // C
GPC = C // L


def _mh_body(x_hbm, out_hbm, xv, acc0, acc1, sem0, sem1):
    c = lax.axis_index("c")
    s = lax.axis_index("s")
    wid = s * NC + c
    base = wid * RW

    stage = pltpu.async_copy(
        x_hbm.at[pl.ds(base * HIST, RW * HIST)], xv, sem0
    )

    zeros = jnp.zeros((L,), jnp.float32)
    ones = jnp.ones((L,), jnp.float32)
    lane_iota = lax.iota(jnp.int32, L)
    lane_pos = lane_iota * HIST
    tailcol = 992 + (lane_iota & 7)

    def _z(r, carry):
        rsplat = lax.broadcast(r, (L,))
        for acc in (acc0, acc1):
            for u in range(VOCAB // L):
                acc[r, pl.ds(u * L, L)] = zeros
            plsc.store_scatter(acc, [rsplat, tailcol], zeros)
        return carry

    lax.fori_loop(0, C, _z, None)
    stage.wait()

    accs = (acc0, acc1)
    sems = (sem0, sem1)

    def scatter_chunk(acc, k, val):
        for g in range(GPC):
            rowv = lane_iota + g * L
            pos0 = lane_pos + (k * C + g * L) * HIST
            cols = [plsc.load_gather(xv, [pos0 + j]) for j in range(HIST)]
            for col in cols:
                if val is None:
                    plsc.addupdate_scatter(acc, [rowv, col], ones)
                else:
                    plsc.store_scatter(acc, [rowv, col], val)

    copies = [None] * NCHUNK
    for k in range(NCHUNK):
        acc = accs[k % 2]
        if k >= 2:
            copies[k - 2].wait()
            scatter_chunk(acc, k - 2, zeros)
        scatter_chunk(acc, k, None)
        copies[k] = pltpu.async_copy(
            acc, out_hbm.at[pl.ds(base + k * C, C), :], sems[k % 2]
        )
    copies[NCHUNK - 2].wait()
    copies[NCHUNK - 1].wait()


def kernel(x):
    x_flat = x.reshape(-1)
    return pl.kernel(
        _mh_body,
        out_type=jax.ShapeDtypeStruct((BATCH, VOCAB), jnp.float32),
        mesh=plsc.VectorSubcoreMesh(core_axis_name="c", subcore_axis_name="s"),
        scratch_types=[
            pltpu.VMEM((RW * HIST,), jnp.int32),
            pltpu.VMEM((C, VOCAB), jnp.float32),
            pltpu.VMEM((C, VOCAB), jnp.float32),
            pltpu.SemaphoreType.DMA,
            pltpu.SemaphoreType.DMA,
        ],
        compiler_params=pltpu.CompilerParams(needs_layout_passes=False),
    )(x_flat)


# speedup vs baseline: 1.4144x; 1.0015x over previous
import jax
import jax.numpy as jnp
from jax import lax
from jax.experimental import pallas as pl
from jax.experimental.pallas import tpu as pltpu
from jax.experimental.pallas import tpu_sc as plsc

VOCAB = 1000
BATCH = 4096
HIST = 20

_info = plsc.get_sparse_core_info()
NC = _info.num_cores
NS = _info.num_subcores
L = _info.num_lanes
NW = NC * NS
RW = BATCH // NW
C = 32
NCHUNK = RW // C
GPC = C // L


def _mh_body(x_hbm, out_hbm, xv, acc0, acc1, sem0, sem1):
    c = lax.axis_index("c")
    s = lax.axis_index("s")
    wid = s * NC + c
    base = wid * RW

    stage = pltpu.async_copy(x_hbm.at[pl.ds(base, RW), :], xv, sem0)

    zeros = jnp.zeros((L,), jnp.float32)
    ones = jnp.ones((L,), jnp.float32)
    lane_iota = lax.iota(jnp.int32, L)
    tailcol = 992 + (lane_iota & 7)

    def _z(r, carry):
        rsplat = lax.broadcast(r, (L,))
        for acc in (acc0, acc1):
            for u in range(VOCAB // L):
                acc[r, pl.ds(u * L, L)] = zeros
            plsc.store_scatter(acc, [rsplat, tailcol], zeros)
        return carry

    lax.fori_loop(0, C, _z, None)
    stage.wait()

    accs = (acc0, acc1)
    sems = (sem0, sem1)

    def scatter_chunk(acc, k, val):
        for g in range(GPC):
            rowv = lane_iota + g * L
            srcrow = lane_iota + (k * C + g * L)
            cols = [
                plsc.load_gather(xv, [srcrow, jnp.full((L,), j, jnp.int32)])
                for j in range(HIST)
            ]
            for col in cols:
                if val is None:
                    plsc.addupdate_scatter(acc, [rowv, col], ones)
                else:
                    plsc.store_scatter(acc, [rowv, col], val)

    copies = [None] * NCHUNK
    for k in range(NCHUNK):
        acc = accs[k % 2]
        if k >= 2:
            copies[k - 2].wait()
            scatter_chunk(acc, k - 2, zeros)
        scatter_chunk(acc, k, None)
        copies[k] = pltpu.async_copy(
            acc, out_hbm.at[pl.ds(base + k * C, C), :], sems[k % 2]
        )
    copies[NCHUNK - 2].wait()
    copies[NCHUNK - 1].wait()


def kernel(x):
    return pl.kernel(
        _mh_body,
        out_type=jax.ShapeDtypeStruct((BATCH, VOCAB), jnp.float32),
        mesh=plsc.VectorSubcoreMesh(core_axis_name="c", subcore_axis_name="s"),
        scratch_types=[
            pltpu.VMEM((RW, HIST), jnp.int32),
            pltpu.VMEM((C, VOCAB), jnp.float32),
            pltpu.VMEM((C, VOCAB), jnp.float32),
            pltpu.SemaphoreType.DMA,
            pltpu.SemaphoreType.DMA,
        ],
        compiler_params=pltpu.CompilerParams(needs_layout_passes=False),
    )(x)


# speedup vs baseline: 1.4177x; 1.0024x over previous
import jax
import jax.numpy as jnp
from jax import lax
from jax.experimental import pallas as pl
from jax.experimental.pallas import tpu as pltpu
from jax.experimental.pallas import tpu_sc as plsc

VOCAB = 1000
BATCH = 4096
HIST = 20

_info = plsc.get_sparse_core_info()
NC = _info.num_cores
NS = _info.num_subcores
L = _info.num_lanes
NW = NC * NS
RW = BATCH // NW
C = 32
NCHUNK = RW // C
GPC = C // L


def _mh_body(x_hbm, out_hbm, xv, acc0, acc1, sem0, sem1):
    c = lax.axis_index("c")
    s = lax.axis_index("s")
    wid = s * NC + c
    base = wid * RW

    stage = pltpu.async_copy(x_hbm.at[pl.ds(base, RW), :], xv, sem0)

    zeros = jnp.zeros((L,), jnp.float32)
    ones = jnp.ones((L,), jnp.float32)
    lane_iota = lax.iota(jnp.int32, L)
    tailcol = 992 + (lane_iota & 7)

    def _z(r, carry):
        rsplat = lax.broadcast(r, (L,))
        for acc in (acc0, acc1):
            for u in range(VOCAB // L):
                acc[r, pl.ds(u * L, L)] = zeros
            plsc.store_scatter(acc, [rsplat, tailcol], zeros)
        return carry

    lax.fori_loop(0, C, _z, None)
    stage.wait()

    accs = (acc0, acc1)
    sems = (sem0, sem1)

    def scatter_chunk(acc, k, val):
        for g in range(GPC):
            rowv = lane_iota + g * L
            srcrow = lane_iota + (k * C + g * L)
            cols = [
                plsc.load_gather(xv, [srcrow, jnp.full((L,), j, jnp.int32)])
                for j in range(HIST)
            ]
            for col in cols:
                if val is None:
                    plsc.addupdate_scatter(acc, [rowv, col], ones)
                else:
                    plsc.store_scatter(acc, [rowv, col], val)

    copies = [None] * NCHUNK
    for k in range(NCHUNK):
        acc = accs[k % 2]
        if k >= 2:
            copies[k - 2].wait()
            scatter_chunk(acc, k - 2, zeros)
        scatter_chunk(acc, k, None)
        copies[k] = pltpu.async_copy(
            acc, out_hbm.at[pl.ds(base + k * C, C), :], sems[k % 2]
        )
    copies[NCHUNK - 2].wait()
    copies[NCHUNK - 1].wait()


def kernel(x):
    return pl.kernel(
        _mh_body,
        out_type=jax.ShapeDtypeStruct((BATCH, VOCAB), jnp.float32),
        mesh=plsc.VectorSubcoreMesh(core_axis_name="c", subcore_axis_name="s"),
        scratch_types=[
            pltpu.VMEM((RW, HIST), jnp.int32),
            pltpu.VMEM((C, VOCAB), jnp.float32),
            pltpu.VMEM((C, VOCAB), jnp.float32),
            pltpu.SemaphoreType.DMA,
            pltpu.SemaphoreType.DMA,
        ],
        compiler_params=pltpu.CompilerParams(
            needs_layout_passes=False, use_tc_tiling_on_sc=True
        ),
    )(x)


# speedup vs baseline: 1.7961x; 1.2669x over previous
import jax
import jax.numpy as jnp
from jax import lax
from jax.experimental import pallas as pl
from jax.experimental.pallas import tpu as pltpu
from jax.experimental.pallas import tpu_sc as plsc

VOCAB = 1000
BATCH = 4096
HIST = 20

_info = plsc.get_sparse_core_info()
NC = _info.num_cores
NS = _info.num_subcores
L = _info.num_lanes
NW = NC * NS
CW = BATCH // NW
GRP = CW // L
VS = 336
NTASK = -(-VOCAB // VS)
SIZES = [min(VS, VOCAB - t * VS) for t in range(NTASK)]


def _mh_body(xT_hbm, outT_hbm, xv, acc0, acc1, sem_x, sem0, sem1):
    c = lax.axis_index("c")
    s = lax.axis_index("s")
    wid = s * NC + c
    base = wid * CW

    stage = pltpu.async_copy(xT_hbm.at[:, pl.ds(base, CW)], xv, sem_x)

    zeros = jnp.zeros((L,), jnp.float32)
    ones = jnp.ones((L,), jnp.float32)
    lane_iota = lax.iota(jnp.int32, L)

    def _z(i, carry):
        for u in range(2):
            for t in range(CW // L):
                acc0[i * 2 + u, pl.ds(t * L, L)] = zeros
                acc1[i * 2 + u, pl.ds(t * L, L)] = zeros
        return carry

    lax.fori_loop(0, VS // 2, _z, None)
    stage.wait()

    accs = (acc0, acc1)
    sems = (sem0, sem1)

    def scatter_task(acc, t, val):
        lo = t * VS
        size = SIZES[t]
        for g in range(GRP):
            colv = lane_iota + g * L
            for j in range(HIST):
                rl = xv[j, pl.ds(g * L, L)] - lo
                m = plsc.bitcast(rl, jnp.uint32) < jnp.uint32(size)
                if val is None:
                    plsc.addupdate_scatter(acc, [rl, colv], ones, mask=m)
                else:
                    plsc.store_scatter(acc, [rl, colv], val, mask=m)

    copies = [None] * NTASK
    for t in range(NTASK):
        acc = accs[t % 2]
        if t >= 2:
            copies[t - 2].wait()
            scatter_task(acc, t - 2, zeros)
        scatter_task(acc, t, None)
        copies[t] = pltpu.async_copy(
            acc.at[pl.ds(0, SIZES[t]), :],
            outT_hbm.at[pl.ds(t * VS, SIZES[t]), pl.ds(base, CW)],
            sems[t % 2],
        )
    copies[NTASK - 2].wait()
    copies[NTASK - 1].wait()


def kernel(x):
    outT = pl.kernel(
        _mh_body,
        out_type=jax.ShapeDtypeStruct((VOCAB, BATCH), jnp.float32),
        mesh=plsc.VectorSubcoreMesh(core_axis_name="c", subcore_axis_name="s"),
        scratch_types=[
            pltpu.VMEM((HIST, CW), jnp.int32),
            pltpu.VMEM((VS, CW), jnp.float32),
            pltpu.VMEM((VS, CW), jnp.float32),
            pltpu.SemaphoreType.DMA,
            pltpu.SemaphoreType.DMA,
            pltpu.SemaphoreType.DMA,
        ],
        compiler_params=pltpu.CompilerParams(
            needs_layout_passes=False, use_tc_tiling_on_sc=True
        ),
    )(x.T)
    return outT.T


# speedup vs baseline: 2.0041x; 1.1158x over previous
import jax
import jax.numpy as jnp
from jax import lax
from jax.experimental import pallas as pl
from jax.experimental.pallas import tpu as pltpu
from jax.experimental.pallas import tpu_sc as plsc

VOCAB = 1000
BATCH = 4096
HIST = 20

_info = plsc.get_sparse_core_info()
NC = _info.num_cores
NS = _info.num_subcores
L = _info.num_lanes
NW = NC * NS
CW = BATCH // NW
GRP = CW // L
VS = 256
NTASK = -(-VOCAB // VS)
SIZES = [min(VS, VOCAB - t * VS) for t in range(NTASK)]


def _mh_body(xT_hbm, outT_hbm, xv, acc0, acc1, sem_x, sem0, sem1):
    c = lax.axis_index("c")
    s = lax.axis_index("s")
    wid = s * NC + c
    base = wid * CW

    stage = pltpu.async_copy(xT_hbm.at[:, pl.ds(base, CW)], xv, sem_x)

    zeros = jnp.zeros((L,), jnp.float32)
    ones = jnp.ones((L,), jnp.float32)
    lane_iota = lax.iota(jnp.int32, L)

    def _z(i, carry):
        for u in range(4):
            for t in range(CW // L):
                acc0[i * 4 + u, pl.ds(t * L, L)] = zeros
                acc1[i * 4 + u, pl.ds(t * L, L)] = zeros
        return carry

    lax.fori_loop(0, VS // 4, _z, None)
    stage.wait()

    accs = (acc0, acc1)
    sems = (sem0, sem1)

    def scatter_task(acc, t, val):
        lo = t * VS
        size = SIZES[t]
        for g in range(GRP):
            colv = lane_iota + g * L
            vs = [xv[j, pl.ds(g * L, L)] for j in range(HIST)]
            rls = [v - lo for v in vs]
            ms = [
                plsc.bitcast(rl, jnp.uint32) < jnp.uint32(size) for rl in rls
            ]
            for rl, m in zip(rls, ms):
                if val is None:
                    plsc.addupdate_scatter(acc, [rl, colv], ones, mask=m)
                else:
                    plsc.store_scatter(acc, [rl, colv], val, mask=m)

    copies = [None] * NTASK
    for t in range(NTASK):
        acc = accs[t % 2]
        if t >= 2:
            copies[t - 2].wait()
            scatter_task(acc, t - 2, zeros)
        scatter_task(acc, t, None)
        copies[t] = pltpu.async_copy(
            acc.at[pl.ds(0, SIZES[t]), :],
            outT_hbm.at[pl.ds(t * VS, SIZES[t]), pl.ds(base, CW)],
            sems[t % 2],
        )
    copies[NTASK - 2].wait()
    copies[NTASK - 1].wait()


def kernel(x):
    outT = pl.kernel(
        _mh_body,
        out_type=jax.ShapeDtypeStruct((VOCAB, BATCH), jnp.float32),
        mesh=plsc.VectorSubcoreMesh(core_axis_name="c", subcore_axis_name="s"),
        scratch_types=[
            pltpu.VMEM((HIST, CW), jnp.int32),
            pltpu.VMEM((VS, CW), jnp.float32),
            pltpu.VMEM((VS, CW), jnp.float32),
            pltpu.SemaphoreType.DMA,
            pltpu.SemaphoreType.DMA,
            pltpu.SemaphoreType.DMA,
        ],
        compiler_params=pltpu.CompilerParams(
            needs_layout_passes=False, use_tc_tiling_on_sc=True
        ),
    )(x.T)
    return outT.T


# speedup vs baseline: 2.2085x; 1.1020x over previous
import jax
import jax.numpy as jnp
from jax import lax
from jax.experimental import pallas as pl
from jax.experimental.pallas import tpu as pltpu
from jax.experimental.pallas import tpu_sc as plsc

VOCAB = 1000
BATCH = 4096
HIST = 20

_info = plsc.get_sparse_core_info()
NC = _info.num_cores
NS = _info.num_subcores
L = _info.num_lanes
NW = NC * NS
CW = BATCH // NW
GRP = CW // L
VS = 160
NTASK = -(-VOCAB // VS)
SIZES = [min(VS, VOCAB - t * VS) for t in range(NTASK)]


def _mh_body(xT_hbm, outT_hbm, xv, acc0, acc1, sem_x, sem0, sem1):
    c = lax.axis_index("c")
    s = lax.axis_index("s")
    wid = s * NC + c
    base = wid * CW

    stage = pltpu.async_copy(xT_hbm.at[:, pl.ds(base, CW)], xv, sem_x)

    zeros = jnp.zeros((L,), jnp.float32)
    ones = jnp.ones((L,), jnp.float32)
    lane_iota = lax.iota(jnp.int32, L)

    def _z(i, carry):
        for u in range(4):
            for t in range(CW // L):
                acc0[i * 4 + u, pl.ds(t * L, L)] = zeros
                acc1[i * 4 + u, pl.ds(t * L, L)] = zeros
        return carry

    lax.fori_loop(0, VS // 4, _z, None)
    stage.wait()

    accs = (acc0, acc1)
    sems = (sem0, sem1)

    def scatter_task(acc, t, val):
        lo = t * VS
        size = SIZES[t]

        def _g(g, carry):
            colv = lane_iota + g * L
            vs = [xv[j, pl.ds(g * L, L)] for j in range(HIST)]
            rls = [v - lo for v in vs]
            ms = [
                plsc.bitcast(rl, jnp.uint32) < jnp.uint32(size) for rl in rls
            ]
            for rl, m in zip(rls, ms):
                if val is None:
                    plsc.addupdate_scatter(acc, [rl, colv], ones, mask=m)
                else:
                    plsc.store_scatter(acc, [rl, colv], val, mask=m)
            return carry

        lax.fori_loop(0, GRP, _g, None)

    copies = [None] * NTASK
    for t in range(NTASK):
        acc = accs[t % 2]
        if t >= 2:
            copies[t - 2].wait()
            scatter_task(acc, t - 2, zeros)
        scatter_task(acc, t, None)
        copies[t] = pltpu.async_copy(
            acc.at[pl.ds(0, SIZES[t]), :],
            outT_hbm.at[pl.ds(t * VS, SIZES[t]), pl.ds(base, CW)],
            sems[t % 2],
        )
    copies[NTASK - 2].wait()
    copies[NTASK - 1].wait()


def kernel(x):
    outT = pl.kernel(
        _mh_body,
        out_type=jax.ShapeDtypeStruct((VOCAB, BATCH), jnp.float32),
        mesh=plsc.VectorSubcoreMesh(core_axis_name="c", subcore_axis_name="s"),
        scratch_types=[
            pltpu.VMEM((HIST, CW), jnp.int32),
            pltpu.VMEM((VS, CW), jnp.float32),
            pltpu.VMEM((VS, CW), jnp.float32),
            pltpu.SemaphoreType.DMA,
            pltpu.SemaphoreType.DMA,
            pltpu.SemaphoreType.DMA,
        ],
        compiler_params=pltpu.CompilerParams(
            needs_layout_passes=False, use_tc_tiling_on_sc=True
        ),
    )(x.T)
    return outT.T


# speedup vs baseline: 2.2240x; 1.0070x over previous
import jax
import jax.numpy as jnp
from jax import lax
from jax.experimental import pallas as pl
from jax.experimental.pallas import tpu as pltpu
from jax.experimental.pallas import tpu_sc as plsc

VOCAB = 1000
BATCH = 4096
HIST = 20

_info = plsc.get_sparse_core_info()
NC = _info.num_cores
NS = _info.num_subcores
L = _info.num_lanes
NW = NC * NS
CW = BATCH // NW
GRP = CW // L
VS = 128
NTASK = -(-VOCAB // VS)
SIZES = [min(VS, VOCAB - t * VS) for t in range(NTASK)]


def _mh_body(xT_hbm, outT_hbm, xv, acc0, acc1, sem_x, sem0, sem1):
    c = lax.axis_index("c")
    s = lax.axis_index("s")
    wid = s * NC + c
    base = wid * CW

    stage = pltpu.async_copy(xT_hbm.at[:, pl.ds(base, CW)], xv, sem_x)

    zeros = jnp.zeros((L,), jnp.float32)
    ones = jnp.ones((L,), jnp.float32)
    lane_iota = lax.iota(jnp.int32, L)

    def zero_buf(acc):
        def _z(i, carry):
            for u in range(4):
                for t in range(CW // L):
                    acc[i * 4 + u, pl.ds(t * L, L)] = zeros
            return carry

        lax.fori_loop(0, VS // 4, _z, None)

    accs = (acc0, acc1)
    sems = (sem0, sem1)

    def scatter_task(acc, t, val):
        lo = t * VS
        size = SIZES[t]

        def _g(g, carry):
            colv = lane_iota + g * L
            vs = [xv[j, pl.ds(g * L, L)] for j in range(HIST)]
            rls = [v - lo for v in vs]
            ms = [
                plsc.bitcast(rl, jnp.uint32) < jnp.uint32(size) for rl in rls
            ]
            for rl, m in zip(rls, ms):
                if val is None:
                    plsc.addupdate_scatter(acc, [rl, colv], ones, mask=m)
                else:
                    plsc.store_scatter(acc, [rl, colv], val, mask=m)
            return carry

        lax.fori_loop(0, GRP, _g, None)

    copies = [None] * NTASK
    for t in range(NTASK):
        acc = accs[t % 2]
        if t < 2:
            zero_buf(acc)
            if t == 0:
                stage.wait()
        else:
            copies[t - 2].wait()
            scatter_task(acc, t - 2, zeros)
        scatter_task(acc, t, None)
        copies[t] = pltpu.async_copy(
            acc.at[pl.ds(0, SIZES[t]), :],
            outT_hbm.at[pl.ds(t * VS, SIZES[t]), pl.ds(base, CW)],
            sems[t % 2],
        )
    copies[NTASK - 2].wait()
    copies[NTASK - 1].wait()


def kernel(x):
    outT = pl.kernel(
        _mh_body,
        out_type=jax.ShapeDtypeStruct((VOCAB, BATCH), jnp.float32),
        mesh=plsc.VectorSubcoreMesh(core_axis_name="c", subcore_axis_name="s"),
        scratch_types=[
            pltpu.VMEM((HIST, CW), jnp.int32),
            pltpu.VMEM((VS, CW), jnp.float32),
            pltpu.VMEM((VS, CW), jnp.float32),
            pltpu.SemaphoreType.DMA,
            pltpu.SemaphoreType.DMA,
            pltpu.SemaphoreType.DMA,
        ],
        compiler_params=pltpu.CompilerParams(
            needs_layout_passes=False, use_tc_tiling_on_sc=True
        ),
    )(x.T)
    return outT.T
